# Initial kernel scaffold; baseline (speedup 1.0000x reference)
import jax
import jax.numpy as jnp
from jax.experimental import pallas as pl


def kernel(x, edge_index, W1, b1, W2, b2, W3, b3, lin1_W, lin1_b, lin2_W, lin2_b):
    raise NotImplementedError("write your pallas kernel here")


# speedup vs baseline: 90.7682x; 90.7682x over previous
import functools

import jax
import jax.numpy as jnp
from jax import lax
from jax.experimental import pallas as pl
from jax.experimental.pallas import tpu as pltpu
from jax.experimental.pallas import tpu_sc as plsc

N = 50000
E = 800000
H = 64

NC = 2
NS = 16
NW = NC * NS
LANES = 16

ROWS = 392
N_PAD = ROWS * 128
E_PAD = 819200
EPT = E_PAD // NW
CHUNK = 5120
NCHUNK = EPT // CHUNK

_MESH = plsc.VectorSubcoreMesh(core_axis_name="c", subcore_axis_name="s")


@functools.partial(
    pl.kernel,
    out_type=jax.ShapeDtypeStruct((NW, N_PAD), jnp.float32),
    mesh=_MESH,
    compiler_params=pltpu.CompilerParams(needs_layout_passes=False),
    scratch_types=[
        pltpu.VMEM((N_PAD,), jnp.float32),
        pltpu.VMEM((N_PAD,), jnp.float32),
        pltpu.VMEM((CHUNK,), jnp.int32),
        pltpu.VMEM((CHUNK,), jnp.int32),
    ],
)
def _sc_pass(src_hbm, dst_hbm, g_hbm, zeros_hbm, out_hbm,
             g_loc, t_loc, sbuf, dbuf):
    wid = lax.axis_index("s") * NC + lax.axis_index("c")
    base = wid * EPT
    pltpu.sync_copy(g_hbm, g_loc)
    pltpu.sync_copy(zeros_hbm, t_loc)
    for c in range(NCHUNK):
        off = base + c * CHUNK
        pltpu.sync_copy(src_hbm.at[pl.ds(off, CHUNK)], sbuf)
        pltpu.sync_copy(dst_hbm.at[pl.ds(off, CHUNK)], dbuf)

        def body(i, carry):
            dvec = dbuf[pl.ds(i * LANES, LANES)]
            svec = sbuf[pl.ds(i * LANES, LANES)]
            vals = plsc.load_gather(g_loc, [dvec])
            plsc.addupdate_scatter(t_loc, [svec], vals)
            return carry

        lax.fori_loop(0, CHUNK // LANES, body, 0)
    pltpu.sync_copy(t_loc, out_hbm.at[wid])


def _dinv_body(parts_ref, dinv_ref):
    deg = jnp.sum(parts_ref[...], axis=0) + 1.0
    dinv = lax.rsqrt(deg)
    dinv = dinv * (1.5 - 0.5 * deg * dinv * dinv)
    dinv = dinv * (1.5 - 0.5 * deg * dinv * dinv)
    idx = (lax.broadcasted_iota(jnp.int32, (ROWS, 128), 0) * 128
           + lax.broadcasted_iota(jnp.int32, (ROWS, 128), 1))
    dinv_ref[...] = jnp.where(idx < N, dinv, 0.0)


def _step_body(parts_ref, dinv_ref, gprev_ref, gnext_ref, s_ref):
    t = jnp.sum(parts_ref[...], axis=0)
    dinv = dinv_ref[...]
    y = dinv * (t + gprev_ref[...])
    gnext_ref[...] = dinv * y
    s_ref[...] = jnp.sum(y).reshape(1, 1)


def _final_body(parts_ref, dinv_ref, gprev_ref, x_ref, su_ref, sv_ref,
                W1_ref, b1_ref, W2_ref, b2_ref, W3_ref, b3_ref,
                l1W_ref, l1b_ref, l2W_ref, l2b_ref, out_ref):
    t = jnp.sum(parts_ref[...], axis=0)
    w = dinv_ref[...] * (t + gprev_ref[...])
    alpha = jnp.sum(w * x_ref[...])
    dot = functools.partial(jnp.dot, precision=lax.Precision.HIGHEST)
    pooled = alpha * W1_ref[...] + sv_ref[0, 0] * b1_ref[...]
    pooled = dot(pooled, W2_ref[...]) + su_ref[0, 0] * b2_ref[...]
    pooled = dot(pooled, W3_ref[...]) + jnp.float32(N) * b3_ref[...]
    h = jnp.maximum(dot(pooled, l1W_ref[...]) + l1b_ref[...], 0.0)
    out_ref[...] = dot(h, l2W_ref[...]) + l2b_ref[...]


def kernel(x, edge_index, W1, b1, W2, b2, W3, b3,
           lin1_W, lin1_b, lin2_W, lin2_b):
    ei = edge_index.astype(jnp.int32)
    pad = jnp.full((E_PAD - E,), N, jnp.int32)
    src = jnp.concatenate([ei[0], pad])
    dst = jnp.concatenate([ei[1], pad])
    zeros = jnp.zeros((N_PAD,), jnp.float32)
    ones = jnp.ones((N_PAD,), jnp.float32)

    deg_parts = _sc_pass(dst, dst, ones, zeros)
    dinv2d = pl.pallas_call(
        _dinv_body,
        out_shape=jax.ShapeDtypeStruct((ROWS, 128), jnp.float32),
    )(deg_parts.reshape(NW, ROWS, 128))
    dinv = dinv2d.reshape(N_PAD)

    step = pl.pallas_call(
        _step_body,
        out_shape=(jax.ShapeDtypeStruct((ROWS, 128), jnp.float32),
                   jax.ShapeDtypeStruct((1, 1), jnp.float32)),
    )
    t1 = _sc_pass(src, dst, dinv, zeros)
    g2_2d, su = step(t1.reshape(NW, ROWS, 128), dinv2d, dinv2d)
    t2 = _sc_pass(src, dst, g2_2d.reshape(N_PAD), zeros)
    g3_2d, sv = step(t2.reshape(NW, ROWS, 128), dinv2d, g2_2d)
    t3 = _sc_pass(src, dst, g3_2d.reshape(N_PAD), zeros)

    x_pad = jnp.pad(x[:, 0], (0, N_PAD - N)).reshape(ROWS, 128)
    out = pl.pallas_call(
        _final_body,
        out_shape=jax.ShapeDtypeStruct((1, 1), jnp.float32),
    )(t3.reshape(NW, ROWS, 128), dinv2d, g3_2d, x_pad, su, sv,
      W1, b1.reshape(1, H), W2, b2.reshape(1, H), W3, b3.reshape(1, H),
      lin1_W, lin1_b.reshape(1, H), lin2_W, lin2_b.reshape(1, 1))
    return out


# speedup vs baseline: 148.6847x; 1.6381x over previous
import functools

import jax
import jax.numpy as jnp
from jax import lax
from jax.experimental import pallas as pl
from jax.experimental.pallas import tpu as pltpu
from jax.experimental.pallas import tpu_sc as plsc

N = 50000
E = 800000
H = 64

NC = 2
NS = 16
NW = NC * NS
LANES = 16

ROWS = 392
N_PAD = ROWS * 128
EPT = E // NW
CHUNK = 5000
NCHUNK = EPT // CHUNK
NVEC = CHUNK // LANES
TAIL = CHUNK - NVEC * LANES
UNROLL = 4
BUF = NVEC * LANES + LANES

_MESH = plsc.VectorSubcoreMesh(core_axis_name="c", subcore_axis_name="s")

_SC_SCRATCH = [
    pltpu.VMEM((N_PAD,), jnp.float32),
    pltpu.VMEM((N_PAD,), jnp.float32),
    pltpu.VMEM((BUF,), jnp.int32),
    pltpu.VMEM((BUF,), jnp.int32),
    pltpu.VMEM((BUF,), jnp.int32),
    pltpu.VMEM((BUF,), jnp.int32),
    pltpu.SemaphoreType.DMA,
    pltpu.SemaphoreType.DMA,
    pltpu.SemaphoreType.DMA,
]


def _sc_pass_body(with_gather, sc_hbm, gt_hbm, g_hbm, out_hbm,
                  g_loc, t_loc, sb0, gb0, sb1, gb1, gsem, sem0, sem1):
    wid = lax.axis_index("s") * NC + lax.axis_index("c")
    base = wid * EPT
    sbufs, gbufs, sems = (sb0, sb1), (gb0, gb1), (sem0, sem1)

    if with_gather:
        gh = pltpu.async_copy(g_hbm, g_loc, gsem)

    def start(c):
        b = c % 2
        off = base + c * CHUNK
        h1 = pltpu.async_copy(sc_hbm.at[pl.ds(off, CHUNK)],
                              sbufs[b].at[pl.ds(0, CHUNK)], sems[b])
        h2 = pltpu.async_copy(gt_hbm.at[pl.ds(off, CHUNK)],
                              gbufs[b].at[pl.ds(0, CHUNK)], sems[b])
        return (h1, h2)

    hs = [None] * NCHUNK
    hs[0] = start(0)

    zv = jnp.zeros((LANES,), jnp.float32)
    ziv = jnp.zeros((LANES,), jnp.int32)
    for b in range(2):
        sbufs[b][pl.ds(NVEC * LANES, LANES)] = ziv
        gbufs[b][pl.ds(NVEC * LANES, LANES)] = ziv

    def zbody(i, carry):
        for k in range(8):
            t_loc[pl.ds(i * 128 + k * LANES, LANES)] = zv
        return carry
    lax.fori_loop(0, N_PAD // 128, zbody, 0)

    if with_gather:
        gh.wait()
    ones = jnp.ones((LANES,), jnp.float32)
    tail_mask = lax.iota(jnp.int32, LANES) < TAIL

    for c in range(NCHUNK):
        b = c % 2
        hs[c][0].wait()
        hs[c][1].wait()
        if c + 1 < NCHUNK:
            hs[c + 1] = start(c + 1)
        sbuf, gbuf = sbufs[b], gbufs[b]

        def body(i, carry):
            for k in range(UNROLL):
                off = i * (UNROLL * LANES) + k * LANES
                svec = sbuf[pl.ds(off, LANES)]
                if with_gather:
                    gvec = gbuf[pl.ds(off, LANES)]
                    vals = plsc.load_gather(g_loc, [gvec])
                else:
                    vals = ones
                plsc.addupdate_scatter(t_loc, [svec], vals)
            return carry
        lax.fori_loop(0, NVEC // UNROLL, body, 0)

        off = NVEC * LANES
        svec = sbuf[pl.ds(off, LANES)]
        if with_gather:
            gvec = gbuf[pl.ds(off, LANES)]
            vals = plsc.load_gather(g_loc, [gvec], mask=tail_mask)
        else:
            vals = ones
        plsc.addupdate_scatter(t_loc, [svec], vals, mask=tail_mask)

    pltpu.sync_copy(t_loc, out_hbm.at[wid])


_sc_spmv = pl.kernel(
    functools.partial(_sc_pass_body, True),
    out_type=jax.ShapeDtypeStruct((NW, N_PAD), jnp.float32),
    mesh=_MESH,
    compiler_params=pltpu.CompilerParams(needs_layout_passes=False),
    scratch_types=_SC_SCRATCH,
)


def _sc_deg_body(dst_hbm, out_hbm, *rest):
    _sc_pass_body(False, dst_hbm, dst_hbm, dst_hbm, out_hbm, *rest)


_sc_deg = pl.kernel(
    _sc_deg_body,
    out_type=jax.ShapeDtypeStruct((NW, N_PAD), jnp.float32),
    mesh=_MESH,
    compiler_params=pltpu.CompilerParams(needs_layout_passes=False),
    scratch_types=_SC_SCRATCH,
)


def _dinv_body(parts_ref, dinv_ref):
    deg = jnp.sum(parts_ref[...], axis=0) + 1.0
    dinv = lax.rsqrt(deg)
    dinv = dinv * (1.5 - 0.5 * deg * dinv * dinv)
    dinv = dinv * (1.5 - 0.5 * deg * dinv * dinv)
    idx = (lax.broadcasted_iota(jnp.int32, (ROWS, 128), 0) * 128
           + lax.broadcasted_iota(jnp.int32, (ROWS, 128), 1))
    dinv_ref[...] = jnp.where(idx < N, dinv, 0.0)


def _step_body(parts_ref, dinv_ref, gprev_ref, gnext_ref, s_ref):
    t = jnp.sum(parts_ref[...], axis=0)
    dinv = dinv_ref[...]
    y = dinv * (t + gprev_ref[...])
    gnext_ref[...] = dinv * y
    s_ref[...] = jnp.sum(y).reshape(1, 1)


def _final_body(parts_ref, dinv_ref, gprev_ref, x_ref, su_ref, sv_ref,
                W1_ref, b1_ref, W2_ref, b2_ref, W3_ref, b3_ref,
                l1W_ref, l1b_ref, l2W_ref, l2b_ref, out_ref):
    t = jnp.sum(parts_ref[...], axis=0)
    w = dinv_ref[...] * (t + gprev_ref[...])
    alpha = jnp.sum(w * x_ref[...])
    dot = functools.partial(jnp.dot, precision=lax.Precision.HIGHEST)
    pooled = alpha * W1_ref[...] + sv_ref[0, 0] * b1_ref[...]
    pooled = dot(pooled, W2_ref[...]) + su_ref[0, 0] * b2_ref[...]
    pooled = dot(pooled, W3_ref[...]) + jnp.float32(N) * b3_ref[...]
    h = jnp.maximum(dot(pooled, l1W_ref[...]) + l1b_ref[...], 0.0)
    out_ref[...] = dot(h, l2W_ref[...]) + l2b_ref[...]


def kernel(x, edge_index, W1, b1, W2, b2, W3, b3,
           lin1_W, lin1_b, lin2_W, lin2_b):
    ei = edge_index.astype(jnp.int32)
    src, dst = ei[0], ei[1]

    deg_parts = _sc_deg(dst)
    dinv2d = pl.pallas_call(
        _dinv_body,
        out_shape=jax.ShapeDtypeStruct((ROWS, 128), jnp.float32),
    )(deg_parts.reshape(NW, ROWS, 128))
    dinv = dinv2d.reshape(N_PAD)

    step = pl.pallas_call(
        _step_body,
        out_shape=(jax.ShapeDtypeStruct((ROWS, 128), jnp.float32),
                   jax.ShapeDtypeStruct((1, 1), jnp.float32)),
    )
    t1 = _sc_spmv(src, dst, dinv)
    g2_2d, su = step(t1.reshape(NW, ROWS, 128), dinv2d, dinv2d)
    t2 = _sc_spmv(src, dst, g2_2d.reshape(N_PAD))
    g3_2d, sv = step(t2.reshape(NW, ROWS, 128), dinv2d, g2_2d)
    t3 = _sc_spmv(src, dst, g3_2d.reshape(N_PAD))

    x_pad = jnp.pad(x[:, 0], (0, N_PAD - N)).reshape(ROWS, 128)
    out = pl.pallas_call(
        _final_body,
        out_shape=jax.ShapeDtypeStruct((1, 1), jnp.float32),
    )(t3.reshape(NW, ROWS, 128), dinv2d, g3_2d, x_pad, su, sv,
      W1, b1.reshape(1, H), W2, b2.reshape(1, H), W3, b3.reshape(1, H),
      lin1_W, lin1_b.reshape(1, H), lin2_W, lin2_b.reshape(1, 1))
    return out
